# baseline (speedup 1.0000x reference)
import functools

import jax
import jax.numpy as jnp
from jax import lax
from jax.experimental import pallas as pl
from jax.experimental.pallas import tpu as pltpu
from jax.experimental.pallas import tpu_sc as plsc

_VOCAB = 1000000
_B = 16384
_H = 50
_D = 32
_NC = 2
_NS = 16
_NW = _NC * _NS
_L = 16

_SW = 512
_NSLAB_FULL = _VOCAB // _SW
_TAIL = _VOCAB - _NSLAB_FULL * _SW
_KMAX = 62

_NB = 256
_UNITS = _H * (_B // _NB)
_UPW = _UNITS // _NW
_UPAIRS = _UPW // 2
_BPW = _UPW * _NB


@functools.partial(
    pl.kernel,
    out_type=jax.ShapeDtypeStruct((_VOCAB // 4, 128), jnp.float32),
    mesh=plsc.VectorSubcoreMesh(core_axis_name="c", subcore_axis_name="s"),
    scratch_types=[
        pltpu.VMEM((_D, _SW), jnp.float32),
        pltpu.VMEM((_D, _SW), jnp.float32),
        pltpu.VMEM((_SW // 4, 128), jnp.float32),
        pltpu.VMEM((_SW // 4, 128), jnp.float32),
        pltpu.SemaphoreType.DMA,
        pltpu.SemaphoreType.DMA,
        pltpu.SemaphoreType.DMA,
        pltpu.SemaphoreType.DMA,
    ],
    compiler_params=pltpu.CompilerParams(needs_layout_passes=False),
)
def _sc_relayout(tblT_hbm, tailT_hbm, out_hbm, s0, s1, t0, t1, i0, i1,
                 o0, o1):
  slab = (s0, s1)
  trans = (t0, t1)
  isem = (i0, i1)
  osem = (o0, o1)
  wid = lax.axis_index("s") * _NC + lax.axis_index("c")

  def slab_id(k):
    return wid + _NW * k

  def transpose_grps(b, glo, ghi):
    def grp_body(i, c):
      ivec = lax.iota(jnp.int32, _L) + i * _L
      lvec = lax.shift_right_logical(ivec, 2)
      cbase = lax.shift_left(lax.bitwise_and(ivec, 3), 5)
      for d in range(_D):
        vals = slab[b][d, pl.ds(i * _L, _L)]
        plsc.store_scatter(trans[b], [lvec, cbase + d], vals)
      return c

    lax.fori_loop(glo, ghi, grp_body, 0)

  def prefetch(k, b):
    s_dyn = slab_id(k)

    @pl.when(s_dyn < _NSLAB_FULL)
    def _():
      pltpu.async_copy(
          tblT_hbm.at[:, pl.ds(s_dyn * _SW, _SW)], slab[b], isem[b])

    @pl.when(s_dyn == _NSLAB_FULL)
    def _():
      pltpu.async_copy(tailT_hbm, slab[b].at[:, pl.ds(0, 128)], isem[b])

  def do_half(k, b):
    s_dyn = slab_id(k)

    @pl.when(s_dyn < _NSLAB_FULL)
    def _():
      pltpu.make_async_copy(
          tblT_hbm.at[:, pl.ds(0, _SW)], slab[b], isem[b]).wait()

      @pl.when(k >= 2)
      def _():
        pltpu.make_async_copy(
            trans[b], out_hbm.at[pl.ds(0, _SW // 4), :], osem[b]).wait()

      transpose_grps(b, 0, _SW // _L)
      pltpu.async_copy(
          trans[b],
          out_hbm.at[pl.ds(s_dyn * (_SW // 4), _SW // 4), :], osem[b])

    @pl.when(s_dyn == _NSLAB_FULL)
    def _():
      pltpu.make_async_copy(
          tailT_hbm, slab[b].at[:, pl.ds(0, 128)], isem[b]).wait()

      @pl.when(k >= 2)
      def _():
        pltpu.make_async_copy(
            trans[b], out_hbm.at[pl.ds(0, _SW // 4), :], osem[b]).wait()

      transpose_grps(b, 4, 8)
      pltpu.async_copy(
          trans[b].at[pl.ds(16, 16)],
          out_hbm.at[pl.ds(_NSLAB_FULL * (_SW // 4), 16), :], osem[b])

  prefetch(0, 0)
  prefetch(1, 1)

  def pair_body(g, carry):
    k0 = 2 * g
    k1 = k0 + 1
    do_half(k0, 0)
    prefetch(k0 + 2, 0)
    do_half(k1, 1)
    prefetch(k1 + 2, 1)
    return carry

  lax.fori_loop(0, (_KMAX - 2) // 2, pair_body, 0)
  do_half(_KMAX - 2, 0)
  do_half(_KMAX - 1, 1)
  @pl.when(slab_id(_KMAX - 2) < _NSLAB_FULL)
  def _():
    pltpu.make_async_copy(
        trans[0], out_hbm.at[pl.ds(0, _SW // 4), :], osem[0]).wait()

  @pl.when(slab_id(_KMAX - 2) == _NSLAB_FULL)
  def _():
    pltpu.make_async_copy(
        trans[0].at[pl.ds(16, 16)],
        out_hbm.at[pl.ds(0, 16), :], osem[0]).wait()

  @pl.when(slab_id(_KMAX - 1) < _NSLAB_FULL)
  def _():
    pltpu.make_async_copy(
        trans[1], out_hbm.at[pl.ds(0, _SW // 4), :], osem[1]).wait()

  @pl.when(slab_id(_KMAX - 1) == _NSLAB_FULL)
  def _():
    pltpu.make_async_copy(
        trans[1].at[pl.ds(16, 16)],
        out_hbm.at[pl.ds(0, 16), :], osem[1]).wait()

  @pl.when(slab_id(_KMAX - 1) > _NSLAB_FULL)
  def _():
    pltpu.make_async_copy(
        trans[1], out_hbm.at[pl.ds(0, _SW // 4), :], osem[1]).wait()


@functools.partial(
    pl.kernel,
    out_type=jax.ShapeDtypeStruct((_H, _D, _B), jnp.float32),
    mesh=plsc.VectorSubcoreMesh(core_axis_name="c", subcore_axis_name="s"),
    scratch_types=[
        pltpu.VMEM((_BPW,), jnp.int32),
        pltpu.VMEM((_NB,), jnp.int32),
        pltpu.VMEM((_NB,), jnp.int32),
        pltpu.VMEM((_NB, 128), jnp.float32),
        pltpu.VMEM((_NB, 128), jnp.float32),
        pltpu.VMEM((_D, _NB), jnp.float32),
        pltpu.VMEM((_D, _NB), jnp.float32),
        pltpu.SemaphoreType.DMA,
        pltpu.SemaphoreType.DMA,
        pltpu.SemaphoreType.DMA,
        pltpu.SemaphoreType.DMA,
        pltpu.SemaphoreType.DMA,
    ],
    compiler_params=pltpu.CompilerParams(needs_layout_passes=False),
)
def _sc_embed(tbl_hbm, idx_hbm, out_hbm, idx_v, l0, l1, r0, r1, t0, t1,
              g0, g1, s0, s1, isem):
  lines = (l0, l1)
  rows = (r0, r1)
  trans = (t0, t1)
  gsem = (g0, g1)
  ssem = (s0, s1)
  wid = lax.axis_index("s") * _NC + lax.axis_index("c")
  gu0 = wid * _UPW

  pltpu.async_copy(idx_hbm.at[pl.ds(gu0 * _NB, _BPW)], idx_v, isem).wait()

  def mk_lines(u, b):
    def grp(i, c):
      v = idx_v[pl.ds(u * _NB + i * _L, _L)]
      lines[b][pl.ds(i * _L, _L)] = jax.lax.shift_right_logical(v, 2)
      return c

    lax.fori_loop(0, _NB // _L, grp, 0)

  def start_gather(b):
    pltpu.async_copy(tbl_hbm.at[lines[b]], rows[b], gsem[b])

  def wait_gather(b):
    pltpu.make_async_copy(tbl_hbm.at[lines[b]], rows[b], gsem[b]).wait()

  def transpose_unit(u, b):
    def grp_body(i, c):
      v = idx_v[pl.ds(u * _NB + i * _L, _L)]
      colbase = lax.shift_left(lax.bitwise_and(v, 3), 5)
      rowvec = lax.iota(jnp.int32, _L) + i * _L
      for d in range(_D):
        vals = plsc.load_gather(rows[b], [rowvec, colbase + d])
        trans[b][d, pl.ds(i * _L, _L)] = vals
      return c

    lax.fori_loop(0, _NB // _L, grp_body, 0)

  def start_store(u, b):
    gu = gu0 + u
    h = gu // (_B // _NB)
    b0 = (gu % (_B // _NB)) * _NB
    pltpu.async_copy(trans[b], out_hbm.at[h, :, pl.ds(b0, _NB)], ssem[b])

  def wait_store(b):
    pltpu.make_async_copy(
        trans[b], out_hbm.at[0, :, pl.ds(0, _NB)], ssem[b]).wait()

  mk_lines(0, 0)
  start_gather(0)

  def pair_body(g, carry):
    u0 = 2 * g
    u1 = u0 + 1
    mk_lines(u1, 1)
    wait_gather(0)
    start_gather(1)

    @pl.when(g > 0)
    def _():
      wait_store(0)

    transpose_unit(u0, 0)
    start_store(u0, 0)
    @pl.when(u1 + 1 < _UPW)
    def _():
      mk_lines(u1 + 1, 0)

    wait_gather(1)

    @pl.when(u1 + 1 < _UPW)
    def _():
      start_gather(0)

    @pl.when(g > 0)
    def _():
      wait_store(1)

    transpose_unit(u1, 1)
    start_store(u1, 1)
    return carry

  lax.fori_loop(0, _UPAIRS, pair_body, 0)
  wait_store(0)
  wait_store(1)


def kernel(inputs, embeddings):
  idx_hmaj = inputs.T.astype(jnp.int32).reshape(_H * _B)
  tblT = embeddings.T
  tailT = jax.lax.slice(tblT, (0, _VOCAB - 128), (_D, _VOCAB))
  tbl = _sc_relayout(tblT, tailT)
  out3 = _sc_embed(tbl, idx_hmaj)
  return out3.transpose(2, 0, 1)


# speedup vs baseline: 2.4722x; 2.4722x over previous
import functools

import jax
import jax.numpy as jnp
from jax import lax
from jax.experimental import pallas as pl
from jax.experimental.pallas import tpu as pltpu
from jax.experimental.pallas import tpu_sc as plsc

_VOCAB = 1000000
_B = 16384
_H = 50
_D = 32
_NC = 2
_NS = 16
_NW = _NC * _NS
_L = 16

_SW = 512
_NSLAB_FULL = _VOCAB // _SW
_TAIL = _VOCAB - _NSLAB_FULL * _SW
_KMAX = 62

_NB = 256
_UNITS = _H * (_B // _NB)
_UPW = _UNITS // _NW
_UPAIRS = _UPW // 2
_BPW = _UPW * _NB


@functools.partial(
    pl.kernel,
    out_type=jax.ShapeDtypeStruct((_VOCAB // 4, 128), jnp.float32),
    mesh=plsc.VectorSubcoreMesh(core_axis_name="c", subcore_axis_name="s"),
    scratch_types=[
        pltpu.VMEM((_D, _SW), jnp.float32),
        pltpu.VMEM((_D, _SW), jnp.float32),
        pltpu.VMEM((_SW // 4, 128), jnp.float32),
        pltpu.VMEM((_SW // 4, 128), jnp.float32),
        pltpu.SemaphoreType.DMA,
        pltpu.SemaphoreType.DMA,
        pltpu.SemaphoreType.DMA,
        pltpu.SemaphoreType.DMA,
    ],
    compiler_params=pltpu.CompilerParams(needs_layout_passes=False),
)
def _sc_relayout(tblT_hbm, tailT_hbm, out_hbm, s0, s1, t0, t1, i0, i1,
                 o0, o1):
  slab = (s0, s1)
  trans = (t0, t1)
  isem = (i0, i1)
  osem = (o0, o1)
  wid = lax.axis_index("s") * _NC + lax.axis_index("c")

  def slab_id(k):
    return wid + _NW * k

  def transpose_grps(b, glo, ghi):
    lanes = lax.iota(jnp.int32, _L)

    def grp_body(i, c):
      ivec = lanes + i * _L
      lvec = lax.shift_right_logical(ivec, 2)
      cbase = lax.shift_left(lax.bitwise_and(ivec, 3), 5)
      for dblk in range(0, _D, _L):
        for k in range(_L):
          dvec = lax.bitwise_and(lanes + k, _L - 1) + dblk
          vals = plsc.load_gather(slab[b], [dvec, ivec])
          plsc.store_scatter(trans[b], [lvec, cbase + dvec], vals)
      return c

    lax.fori_loop(glo, ghi, grp_body, 0)

  def prefetch(k, b):
    s_dyn = slab_id(k)

    @pl.when(s_dyn < _NSLAB_FULL)
    def _():
      pltpu.async_copy(
          tblT_hbm.at[:, pl.ds(s_dyn * _SW, _SW)], slab[b], isem[b])

    @pl.when(s_dyn == _NSLAB_FULL)
    def _():
      pltpu.async_copy(tailT_hbm, slab[b].at[:, pl.ds(0, 128)], isem[b])

  def do_half(k, b):
    s_dyn = slab_id(k)

    @pl.when(s_dyn < _NSLAB_FULL)
    def _():
      pltpu.make_async_copy(
          tblT_hbm.at[:, pl.ds(0, _SW)], slab[b], isem[b]).wait()

      @pl.when(k >= 2)
      def _():
        pltpu.make_async_copy(
            trans[b], out_hbm.at[pl.ds(0, _SW // 4), :], osem[b]).wait()

      transpose_grps(b, 0, _SW // _L)
      pltpu.async_copy(
          trans[b],
          out_hbm.at[pl.ds(s_dyn * (_SW // 4), _SW // 4), :], osem[b])

    @pl.when(s_dyn == _NSLAB_FULL)
    def _():
      pltpu.make_async_copy(
          tailT_hbm, slab[b].at[:, pl.ds(0, 128)], isem[b]).wait()

      @pl.when(k >= 2)
      def _():
        pltpu.make_async_copy(
            trans[b], out_hbm.at[pl.ds(0, _SW // 4), :], osem[b]).wait()

      transpose_grps(b, 4, 8)
      pltpu.async_copy(
          trans[b].at[pl.ds(16, 16)],
          out_hbm.at[pl.ds(_NSLAB_FULL * (_SW // 4), 16), :], osem[b])

  prefetch(0, 0)
  prefetch(1, 1)

  def pair_body(g, carry):
    k0 = 2 * g
    k1 = k0 + 1
    do_half(k0, 0)
    prefetch(k0 + 2, 0)
    do_half(k1, 1)
    prefetch(k1 + 2, 1)
    return carry

  lax.fori_loop(0, (_KMAX - 2) // 2, pair_body, 0)
  do_half(_KMAX - 2, 0)
  do_half(_KMAX - 1, 1)
  @pl.when(slab_id(_KMAX - 2) < _NSLAB_FULL)
  def _():
    pltpu.make_async_copy(
        trans[0], out_hbm.at[pl.ds(0, _SW // 4), :], osem[0]).wait()

  @pl.when(slab_id(_KMAX - 2) == _NSLAB_FULL)
  def _():
    pltpu.make_async_copy(
        trans[0].at[pl.ds(16, 16)],
        out_hbm.at[pl.ds(0, 16), :], osem[0]).wait()

  @pl.when(slab_id(_KMAX - 1) < _NSLAB_FULL)
  def _():
    pltpu.make_async_copy(
        trans[1], out_hbm.at[pl.ds(0, _SW // 4), :], osem[1]).wait()

  @pl.when(slab_id(_KMAX - 1) == _NSLAB_FULL)
  def _():
    pltpu.make_async_copy(
        trans[1].at[pl.ds(16, 16)],
        out_hbm.at[pl.ds(0, 16), :], osem[1]).wait()

  @pl.when(slab_id(_KMAX - 1) > _NSLAB_FULL)
  def _():
    pltpu.make_async_copy(
        trans[1], out_hbm.at[pl.ds(0, _SW // 4), :], osem[1]).wait()


@functools.partial(
    pl.kernel,
    out_type=jax.ShapeDtypeStruct((_H, _D, _B), jnp.float32),
    mesh=plsc.VectorSubcoreMesh(core_axis_name="c", subcore_axis_name="s"),
    scratch_types=[
        pltpu.VMEM((_BPW,), jnp.int32),
        pltpu.VMEM((_NB,), jnp.int32),
        pltpu.VMEM((_NB,), jnp.int32),
        pltpu.VMEM((_NB, 128), jnp.float32),
        pltpu.VMEM((_NB, 128), jnp.float32),
        pltpu.VMEM((_D, _NB), jnp.float32),
        pltpu.VMEM((_D, _NB), jnp.float32),
        pltpu.SemaphoreType.DMA,
        pltpu.SemaphoreType.DMA,
        pltpu.SemaphoreType.DMA,
        pltpu.SemaphoreType.DMA,
        pltpu.SemaphoreType.DMA,
    ],
    compiler_params=pltpu.CompilerParams(needs_layout_passes=False),
)
def _sc_embed(tbl_hbm, idx_hbm, out_hbm, idx_v, l0, l1, r0, r1, t0, t1,
              g0, g1, s0, s1, isem):
  lines = (l0, l1)
  rows = (r0, r1)
  trans = (t0, t1)
  gsem = (g0, g1)
  ssem = (s0, s1)
  wid = lax.axis_index("s") * _NC + lax.axis_index("c")
  gu0 = wid * _UPW

  pltpu.async_copy(idx_hbm.at[pl.ds(gu0 * _NB, _BPW)], idx_v, isem).wait()

  def mk_lines(u, b):
    def grp(i, c):
      v = idx_v[pl.ds(u * _NB + i * _L, _L)]
      lines[b][pl.ds(i * _L, _L)] = jax.lax.shift_right_logical(v, 2)
      return c

    lax.fori_loop(0, _NB // _L, grp, 0)

  def start_gather(b):
    pltpu.async_copy(tbl_hbm.at[lines[b]], rows[b], gsem[b])

  def wait_gather(b):
    pltpu.make_async_copy(tbl_hbm.at[lines[b]], rows[b], gsem[b]).wait()

  def transpose_unit(u, b):
    lanes = lax.iota(jnp.int32, _L)

    def grp_body(i, c):
      v = idx_v[pl.ds(u * _NB + i * _L, _L)]
      colbase = lax.shift_left(lax.bitwise_and(v, 3), 5)
      rowvec = lanes + i * _L
      for dblk in range(0, _D, _L):
        for k in range(_L):
          dvec = lax.bitwise_and(lanes + k, _L - 1) + dblk
          vals = plsc.load_gather(rows[b], [rowvec, colbase + dvec])
          plsc.store_scatter(trans[b], [dvec, rowvec], vals)
      return c

    lax.fori_loop(0, _NB // _L, grp_body, 0)

  def start_store(u, b):
    gu = gu0 + u
    h = gu // (_B // _NB)
    b0 = (gu % (_B // _NB)) * _NB
    pltpu.async_copy(trans[b], out_hbm.at[h, :, pl.ds(b0, _NB)], ssem[b])

  def wait_store(b):
    pltpu.make_async_copy(
        trans[b], out_hbm.at[0, :, pl.ds(0, _NB)], ssem[b]).wait()

  mk_lines(0, 0)
  start_gather(0)

  def pair_body(g, carry):
    u0 = 2 * g
    u1 = u0 + 1
    mk_lines(u1, 1)
    wait_gather(0)
    start_gather(1)

    @pl.when(g > 0)
    def _():
      wait_store(0)

    transpose_unit(u0, 0)
    start_store(u0, 0)
    @pl.when(u1 + 1 < _UPW)
    def _():
      mk_lines(u1 + 1, 0)

    wait_gather(1)

    @pl.when(u1 + 1 < _UPW)
    def _():
      start_gather(0)

    @pl.when(g > 0)
    def _():
      wait_store(1)

    transpose_unit(u1, 1)
    start_store(u1, 1)
    return carry

  lax.fori_loop(0, _UPAIRS, pair_body, 0)
  wait_store(0)
  wait_store(1)


def kernel(inputs, embeddings):
  idx_hmaj = inputs.T.astype(jnp.int32).reshape(_H * _B)
  tblT = embeddings.T
  tailT = jax.lax.slice(tblT, (0, _VOCAB - 128), (_D, _VOCAB))
  tbl = _sc_relayout(tblT, tailT)
  out3 = _sc_embed(tbl, idx_hmaj)
  return out3.transpose(2, 0, 1)
